# baseline (speedup 1.0000x reference)
import functools

import jax
import jax.numpy as jnp
from jax import lax
from jax.experimental import pallas as pl
from jax.experimental.pallas import tpu as pltpu
from jax.experimental.pallas import tpu_sc as plsc

_NUM_CORES = 2
_NUM_SUBCORES = 16
_NW = _NUM_CORES * _NUM_SUBCORES
_CHUNK = 128


@jax.jit
def _embed_lookup(idx2, table):
    n_rows, chunk = idx2.shape
    n_total = n_rows * chunk
    _, d = table.shape
    rows_per_w = n_rows // _NW
    b_per_w = rows_per_w * chunk

    mesh = plsc.VectorSubcoreMesh(core_axis_name="c", subcore_axis_name="s")

    @functools.partial(
        pl.kernel,
        out_type=jax.ShapeDtypeStruct((n_total, d), jnp.float32),
        mesh=mesh,
        compiler_params=pltpu.CompilerParams(use_tc_tiling_on_sc=False),
        scratch_types=[
            pltpu.VMEM((rows_per_w, chunk), jnp.int32),
            pltpu.VMEM((chunk, d), jnp.float32),
            pltpu.SemaphoreType.DMA,
        ],
    )
    def body(idx_hbm, table_hbm, out_hbm, idx_v, rows_v, sem):
        wid = lax.axis_index("s") * _NUM_CORES + lax.axis_index("c")
        row_base = wid * rows_per_w
        out_base = wid * b_per_w

        pltpu.sync_copy(idx_hbm.at[pl.ds(row_base, rows_per_w)], idx_v)

        def step(j, carry):
            pltpu.async_copy(table_hbm.at[idx_v.at[j]], rows_v, sem).wait()
            pltpu.sync_copy(rows_v, out_hbm.at[pl.ds(out_base + j * chunk, chunk)])
            return carry

        lax.fori_loop(0, rows_per_w, step, 0)

    return body(idx2, table)


def kernel(x, table):
    b, s = x.shape
    idx = x.reshape(-1).astype(jnp.int32)
    idx2 = idx.reshape(idx.size // _CHUNK, _CHUNK)
    out = _embed_lookup(idx2, table)
    return out.reshape(b, s, table.shape[1])


# speedup vs baseline: 1.1113x; 1.1113x over previous
import functools

import jax
import jax.numpy as jnp
from jax import lax
from jax.experimental import pallas as pl
from jax.experimental.pallas import tpu as pltpu
from jax.experimental.pallas import tpu_sc as plsc

_NUM_CORES = 2
_NUM_SUBCORES = 16
_NW = _NUM_CORES * _NUM_SUBCORES
_CHUNK = 128
_NBUF = 4


@jax.jit
def _embed_lookup(idx2, table):
    n_rows, chunk = idx2.shape
    n_total = n_rows * chunk
    _, d = table.shape
    rows_per_w = n_rows // _NW
    b_per_w = rows_per_w * chunk
    n_groups = rows_per_w // _NBUF

    mesh = plsc.VectorSubcoreMesh(core_axis_name="c", subcore_axis_name="s")

    @functools.partial(
        pl.kernel,
        out_type=jax.ShapeDtypeStruct((n_total, d), jnp.float32),
        mesh=mesh,
        compiler_params=pltpu.CompilerParams(use_tc_tiling_on_sc=False),
        scratch_types=(
            [pltpu.VMEM((rows_per_w, chunk), jnp.int32)]
            + [pltpu.VMEM((chunk, d), jnp.float32)] * _NBUF
            + [pltpu.SemaphoreType.DMA] * (2 * _NBUF)
        ),
    )
    def body(idx_hbm, table_hbm, out_hbm, idx_v, *rest):
        bufs = rest[:_NBUF]
        gsems = rest[_NBUF : 2 * _NBUF]
        wsems = rest[2 * _NBUF :]

        wid = lax.axis_index("s") * _NUM_CORES + lax.axis_index("c")
        row_base = wid * rows_per_w
        out_base = wid * b_per_w

        pltpu.sync_copy(idx_hbm.at[pl.ds(row_base, rows_per_w)], idx_v)

        def gather(j, b):
            pltpu.async_copy(table_hbm.at[idx_v.at[j]], bufs[b], gsems[b])

        def gather_wait(j, b):
            pltpu.make_async_copy(
                table_hbm.at[idx_v.at[j]], bufs[b], gsems[b]
            ).wait()

        def wb(j, b):
            pltpu.async_copy(
                bufs[b], out_hbm.at[pl.ds(out_base + j * chunk, chunk)], wsems[b]
            )

        def wb_wait(j, b):
            pltpu.make_async_copy(
                bufs[b], out_hbm.at[pl.ds(out_base + j * chunk, chunk)], wsems[b]
            ).wait()

        for b in range(_NBUF):
            gather(b, b)

        def group(g, carry):
            j0 = g * _NBUF
            for b in range(_NBUF):
                gather_wait(j0 + b, b)
                wb(j0 + b, b)
            for b in range(_NBUF):
                wb_wait(j0 + b, b)
                gather(j0 + _NBUF + b, b)
            return carry

        lax.fori_loop(0, n_groups - 1, group, 0)

        j0 = (n_groups - 1) * _NBUF
        for b in range(_NBUF):
            gather_wait(j0 + b, b)
            wb(j0 + b, b)
        for b in range(_NBUF):
            wb_wait(j0 + b, b)

    return body(idx2, table)


def kernel(x, table):
    b, s = x.shape
    idx = x.reshape(-1).astype(jnp.int32)
    idx2 = idx.reshape(idx.size // _CHUNK, _CHUNK)
    out = _embed_lookup(idx2, table)
    return out.reshape(b, s, table.shape[1])


# speedup vs baseline: 1.5866x; 1.4277x over previous
import functools

import jax
import jax.numpy as jnp
from jax import lax
from jax.experimental import pallas as pl
from jax.experimental.pallas import tpu as pltpu
from jax.experimental.pallas import tpu_sc as plsc

_NUM_CORES = 2
_NUM_SUBCORES = 16
_NW = _NUM_CORES * _NUM_SUBCORES
_CHUNK = 128
_NBUF = 4

_PACK_BLK = 8192


def _pack_body(t_ref, o_ref):
    v = t_ref[...]
    vt = v.T
    o_ref[...] = jnp.concatenate([vt, vt], axis=1)


@jax.jit
def _pack_table(t_t):
    d, n = t_t.shape
    grid = (n + _PACK_BLK - 1) // _PACK_BLK
    return pl.pallas_call(
        _pack_body,
        grid=(grid,),
        in_specs=[pl.BlockSpec((d, _PACK_BLK), lambda i: (0, i))],
        out_specs=pl.BlockSpec((_PACK_BLK, 128), lambda i: (i, 0)),
        out_shape=jax.ShapeDtypeStruct((n, 128), jnp.float32),
    )(t_t)


@jax.jit
def _gather2(idx2, t2):
    n_rows, chunk = idx2.shape
    n_total = n_rows * chunk
    rows_per_w = n_rows // _NW
    b_per_w = rows_per_w * chunk
    n_groups = rows_per_w // _NBUF

    mesh = plsc.VectorSubcoreMesh(core_axis_name="c", subcore_axis_name="s")

    @functools.partial(
        pl.kernel,
        out_type=jax.ShapeDtypeStruct((n_total, 128), jnp.float32),
        mesh=mesh,
        compiler_params=pltpu.CompilerParams(use_tc_tiling_on_sc=False),
        scratch_types=(
            [pltpu.VMEM((rows_per_w, chunk), jnp.int32)]
            + [pltpu.VMEM((chunk, 128), jnp.float32)] * _NBUF
            + [pltpu.SemaphoreType.DMA] * (2 * _NBUF)
        ),
    )
    def body(idx_hbm, t_hbm, out_hbm, idx_v, *rest):
        bufs = rest[:_NBUF]
        gsems = rest[_NBUF : 2 * _NBUF]
        wsems = rest[2 * _NBUF :]

        wid = lax.axis_index("s") * _NUM_CORES + lax.axis_index("c")
        row_base = wid * rows_per_w
        out_base = wid * b_per_w

        pltpu.sync_copy(idx_hbm.at[pl.ds(row_base, rows_per_w)], idx_v)

        def gather(j, b):
            pltpu.async_copy(t_hbm.at[idx_v.at[j]], bufs[b], gsems[b])

        def gather_wait(j, b):
            pltpu.make_async_copy(t_hbm.at[idx_v.at[j]], bufs[b], gsems[b]).wait()

        def wb(j, b):
            pltpu.async_copy(
                bufs[b],
                out_hbm.at[pl.ds(out_base + j * chunk, chunk)],
                wsems[b],
            )

        def wb_wait(j, b):
            pltpu.make_async_copy(
                bufs[b],
                out_hbm.at[pl.ds(out_base + j * chunk, chunk)],
                wsems[b],
            ).wait()

        for b in range(_NBUF):
            gather(b, b)

        def group(g, carry):
            j0 = g * _NBUF
            for b in range(_NBUF):
                gather_wait(j0 + b, b)
                wb(j0 + b, b)
            for b in range(_NBUF):
                wb_wait(j0 + b, b)
                gather(j0 + _NBUF + b, b)
            return carry

        lax.fori_loop(0, n_groups - 1, group, 0)

        j0 = (n_groups - 1) * _NBUF
        for b in range(_NBUF):
            gather_wait(j0 + b, b)
            wb(j0 + b, b)
        for b in range(_NBUF):
            wb_wait(j0 + b, b)

    return body(idx2, t2)


_UNPACK_BLK = 128


def _unpack_body(g_ref, o_ref):
    g = g_ref[...]
    f = g[:, :64]
    f3 = f.reshape(_UNPACK_BLK, 200, 64)
    o_ref[...] = f3.transpose(1, 2, 0)


@functools.partial(jax.jit, static_argnums=(1, 2, 3))
def _unpack(g2, b, s, d):
    n_total = g2.shape[0]
    rows_blk = _UNPACK_BLK * s
    grid = n_total // rows_blk
    return pl.pallas_call(
        _unpack_body,
        grid=(grid,),
        in_specs=[pl.BlockSpec((rows_blk, 128), lambda i: (i, 0))],
        out_specs=pl.BlockSpec((s, d, _UNPACK_BLK), lambda i: (0, 0, i)),
        out_shape=jax.ShapeDtypeStruct((s, d, b), jnp.float32),
    )(g2)


def kernel(x, table):
    b, s = x.shape
    _, d = table.shape
    idx = x.reshape(-1).astype(jnp.int32)
    idx2 = idx.reshape(idx.size // _CHUNK, _CHUNK)
    t2 = _pack_table(table.T)
    g2 = _gather2(idx2, t2)
    out3 = _unpack(g2, b, s, d)
    return out3.transpose(2, 0, 1)
